# Initial kernel scaffold; baseline (speedup 1.0000x reference)
import jax
import jax.numpy as jnp
from jax.experimental import pallas as pl


def kernel(input_ids, word_emb, pos_emb, type_emb, ln_w, ln_b):
    raise NotImplementedError("write your pallas kernel here")


# speedup vs baseline: 1.7670x; 1.7670x over previous
import functools

import jax
import jax.numpy as jnp
from jax import lax
from jax.experimental import pallas as pl
from jax.experimental.pallas import tpu as pltpu
from jax.experimental.pallas import tpu_sc as plsc

_PAD = 1
_EPS = 1e-05
_HIDDEN = 768

_NC = 2
_NS = 16
_NW = _NC * _NS
_CH = 64


def _posid_body(ids_ref, out_ref):
    ids = ids_ref[...]
    mask = (ids != _PAD).astype(jnp.int32)
    x = mask
    seq = ids.shape[1]
    k = 1
    while k < seq:
        shifted = jnp.concatenate(
            [jnp.zeros((ids.shape[0], k), jnp.int32), x[:, :-k]], axis=1)
        x = x + shifted
        k *= 2
    out_ref[...] = x * mask + _PAD


def _position_ids(input_ids):
    return pl.pallas_call(
        _posid_body,
        out_shape=jax.ShapeDtypeStruct(input_ids.shape, jnp.int32),
    )(input_ids)


@functools.lru_cache(maxsize=None)
def _make_gather(v_word, v_pos, d, b):
    rpw = b // _NW
    nch = rpw // _CH
    mesh = plsc.VectorSubcoreMesh(core_axis_name="c", subcore_axis_name="s")

    @functools.partial(
        pl.kernel,
        mesh=mesh,
        out_type=[
            jax.ShapeDtypeStruct((b, d), jnp.float32),
            jax.ShapeDtypeStruct((b, d), jnp.float32),
        ],
        scratch_types=[
            pltpu.VMEM((_CH,), jnp.int32),
            pltpu.VMEM((_CH,), jnp.int32),
            pltpu.VMEM((_CH, d), jnp.float32),
            pltpu.VMEM((_CH, d), jnp.float32),
            pltpu.SemaphoreType.DMA,
            pltpu.SemaphoreType.DMA,
        ],
    )
    def gather_kernel(word_hbm, pos_hbm, iw_hbm, ip_hbm, outw_hbm, outp_hbm,
                      iw_v, ip_v, wbuf, pbuf, semw, semp):
        wid = lax.axis_index("s") * _NC + lax.axis_index("c")
        base = wid * rpw

        @pl.loop(0, nch)
        def _(c):
            off = base + c * _CH
            pltpu.sync_copy(iw_hbm.at[pl.ds(off, _CH)], iw_v)
            pltpu.sync_copy(ip_hbm.at[pl.ds(off, _CH)], ip_v)
            cw = pltpu.async_copy(word_hbm.at[iw_v], wbuf, semw)
            cp = pltpu.async_copy(pos_hbm.at[ip_v], pbuf, semp)
            cw.wait()
            cp.wait()
            pltpu.sync_copy(wbuf, outw_hbm.at[pl.ds(off, _CH)])
            pltpu.sync_copy(pbuf, outp_hbm.at[pl.ds(off, _CH)])

    return gather_kernel


def _ln_body(w_ref, p_ref, t_ref, lw_ref, lb_ref, o_ref):
    x = w_ref[...] + p_ref[...] + t_ref[0:1, :]
    mean = jnp.mean(x, axis=-1, keepdims=True)
    xc = x - mean
    var = jnp.mean(xc * xc, axis=-1, keepdims=True)
    o_ref[...] = xc * lax.rsqrt(var + _EPS) * lw_ref[...] + lb_ref[...]


def _ln(w_rows, p_rows, type_emb, ln_w, ln_b):
    b, d = w_rows.shape
    rb = 512
    grid = (b // rb,)
    return pl.pallas_call(
        _ln_body,
        grid=grid,
        in_specs=[
            pl.BlockSpec((rb, d), lambda i: (i, 0)),
            pl.BlockSpec((rb, d), lambda i: (i, 0)),
            pl.BlockSpec(type_emb.shape, lambda i: (0, 0)),
            pl.BlockSpec((1, d), lambda i: (0, 0)),
            pl.BlockSpec((1, d), lambda i: (0, 0)),
        ],
        out_specs=pl.BlockSpec((rb, d), lambda i: (i, 0)),
        out_shape=jax.ShapeDtypeStruct((b, d), jnp.float32),
    )(w_rows, p_rows, type_emb, ln_w, ln_b)


def kernel(input_ids, word_emb, pos_emb, type_emb, ln_w, ln_b):
    bb, seq = input_ids.shape
    d = word_emb.shape[1]
    b = bb * seq

    position_ids = _position_ids(input_ids)
    ids_flat = input_ids.reshape(b)
    pos_flat = position_ids.reshape(b)

    gather = _make_gather(word_emb.shape[0], pos_emb.shape[0], d, b)
    w_rows, p_rows = gather(word_emb, pos_emb, ids_flat, pos_flat)

    out = _ln(w_rows, p_rows, type_emb,
              ln_w.reshape(1, d), ln_b.reshape(1, d))
    return out.reshape(bb, seq, d)
